# baseline (speedup 1.0000x reference)
import functools

import jax
import jax.numpy as jnp
from jax import lax
from jax.experimental import pallas as pl
from jax.experimental.pallas import tpu as pltpu
from jax.experimental.pallas import tpu_sc as plsc

_NC = 2
_NS = 16
_CE = 16384


def _sc_body(total_elems, pe_elems, seq_hbm, pe_hbm, out_hbm,
             a0, a1, b0, b1, sp0, sp1, ss0, ss1, so0, so1):
    nw = _NC * _NS
    elems_per_w = total_elems // nw
    n_chunks = elems_per_w // _CE
    wid = lax.axis_index("s") * _NC + lax.axis_index("c")
    e0 = wid * elems_per_w
    p0 = lax.rem(e0, pe_elems)

    pe_bufs = (a0, a1)
    seq_bufs = (b0, b1)
    sems_pe = (sp0, sp1)
    sems_seq = (ss0, ss1)
    sems_out = (so0, so1)

    def start_loads(c, b):
        off = c * _CE
        hp = pltpu.async_copy(pe_hbm.at[pl.ds(p0 + off, _CE)], pe_bufs[b],
                              sems_pe[b])
        hs = pltpu.async_copy(seq_hbm.at[pl.ds(e0 + off, _CE)], seq_bufs[b],
                              sems_seq[b])
        return hp, hs

    load_handles = [None, None]
    out_handles = [None, None]
    load_handles[0] = start_loads(0, 0)
    for c in range(n_chunks):
        b = c & 1
        nb = 1 - b
        if c + 1 < n_chunks:
            if out_handles[nb] is not None:
                out_handles[nb].wait()
            load_handles[nb] = start_loads(c + 1, nb)
        hp, hs = load_handles[b]
        hp.wait()
        hs.wait()

        pe_buf = pe_bufs[b]
        seq_buf = seq_bufs[b]

        @plsc.parallel_loop(0, _CE, step=16, unroll=8)
        def _add(i):
            pe_buf[pl.ds(i, 16)] = pe_buf[pl.ds(i, 16)] + seq_buf[pl.ds(i, 16)]

        out_handles[b] = pltpu.async_copy(
            pe_buf, out_hbm.at[pl.ds(e0 + c * _CE, _CE)], sems_out[b])
    for h in out_handles:
        if h is not None:
            h.wait()


def kernel(sequence, pe):
    B, S, D = sequence.shape
    total = B * S * D
    flat = sequence.reshape(total)

    sc_add = pl.kernel(
        functools.partial(_sc_body, total, S * D),
        out_type=jax.ShapeDtypeStruct((total,), jnp.float32),
        mesh=plsc.VectorSubcoreMesh(
            core_axis_name="c", subcore_axis_name="s",
            num_cores=_NC, num_subcores=_NS),
        scratch_types=[
            pltpu.VMEM((_CE,), jnp.float32),
            pltpu.VMEM((_CE,), jnp.float32),
            pltpu.VMEM((_CE,), jnp.float32),
            pltpu.VMEM((_CE,), jnp.float32),
            pltpu.SemaphoreType.DMA,
            pltpu.SemaphoreType.DMA,
            pltpu.SemaphoreType.DMA,
            pltpu.SemaphoreType.DMA,
            pltpu.SemaphoreType.DMA,
            pltpu.SemaphoreType.DMA,
        ],
    )
    out = sc_add(flat, pe[:S].reshape(S * D))
    return out.reshape(B, S, D)


# speedup vs baseline: 3.2050x; 3.2050x over previous
import functools

import jax
import jax.numpy as jnp
from jax import lax
from jax.experimental import pallas as pl
from jax.experimental.pallas import tpu as pltpu
from jax.experimental.pallas import tpu_sc as plsc

_B = 4
_S = 8192
_D = 1024
_ROWS = _B * _S
_TC1_ROWS = 24576
_TC2_ROWS = 4096
_SC_ROW0 = _TC1_ROWS + _TC2_ROWS
_BS_A = 2048
_BS_B = 1024
_NC = 2
_NS = 16
_CR = 16


def _tc_add_block(seq_ref, pe_ref, out_ref):
    out_ref[...] = seq_ref[...] + pe_ref[...]


def _tc_add_block_alias(seq_ref, pe_ref, _aliased_ref, out_ref):
    out_ref[...] = seq_ref[...] + pe_ref[...]


def _sc_body(seq_hbm, pe_hbm, out_hbm):
    def inner(a0, a1, b0, b1, sp0, sp1, ss0, ss1, so0, so1):
        nw = _NC * _NS
        rows_per_w = (_ROWS - _SC_ROW0) // nw
        n_chunks = rows_per_w // _CR
        wid = lax.axis_index("s") * _NC + lax.axis_index("c")
        o0 = wid * rows_per_w
        r0 = _SC_ROW0 + o0
        p0 = lax.rem(r0, _S)

        pe_bufs = (a0, a1)
        seq_bufs = (b0, b1)
        sems_pe = (sp0, sp1)
        sems_seq = (ss0, ss1)
        sems_out = (so0, so1)

        def start_loads(c, b):
            hp = pltpu.async_copy(pe_hbm.at[pl.ds(p0 + c * _CR, _CR)],
                                  pe_bufs[b], sems_pe[b])
            hs = pltpu.async_copy(seq_hbm.at[pl.ds(r0 + c * _CR, _CR)],
                                  seq_bufs[b], sems_seq[b])
            return hp, hs

        load_handles = [None, None]
        out_handles = [None, None]
        load_handles[0] = start_loads(0, 0)
        for c in range(n_chunks):
            b = c & 1
            nb = 1 - b
            if c + 1 < n_chunks:
                if out_handles[nb] is not None:
                    out_handles[nb].wait()
                load_handles[nb] = start_loads(c + 1, nb)
            hp, hs = load_handles[b]
            hp.wait()
            hs.wait()

            pe_buf = pe_bufs[b]
            seq_buf = seq_bufs[b]

            @plsc.parallel_loop(0, _CR * _D, step=16, unroll=8)
            def _add(i):
                r = i >> 10
                o = pl.multiple_of(i & (_D - 1), 16)
                pe_buf[r, pl.ds(o, 16)] = (
                    pe_buf[r, pl.ds(o, 16)] + seq_buf[r, pl.ds(o, 16)])

            out_handles[b] = pltpu.async_copy(
                pe_buf, out_hbm.at[pl.ds(o0 + c * _CR, _CR)], sems_out[b])
        for h in out_handles:
            if h is not None:
                h.wait()

    pl.run_scoped(
        inner,
        pltpu.VMEM((_CR, _D), jnp.float32),
        pltpu.VMEM((_CR, _D), jnp.float32),
        pltpu.VMEM((_CR, _D), jnp.float32),
        pltpu.VMEM((_CR, _D), jnp.float32),
        pltpu.SemaphoreType.DMA,
        pltpu.SemaphoreType.DMA,
        pltpu.SemaphoreType.DMA,
        pltpu.SemaphoreType.DMA,
        pltpu.SemaphoreType.DMA,
        pltpu.SemaphoreType.DMA,
    )


def kernel(sequence, pe):
    B, S, D = sequence.shape
    flat = sequence.reshape(B * S, D)
    pe_s = pe[:S]

    sc_tail = pl.kernel(
        _sc_body,
        out_type=jax.ShapeDtypeStruct((_ROWS - _SC_ROW0, D), jnp.float32),
        mesh=plsc.VectorSubcoreMesh(
            core_axis_name="c", subcore_axis_name="s",
            num_cores=_NC, num_subcores=_NS),
    )
    sc_out = sc_tail(flat, pe_s)

    nsb = S // _BS_A
    buf1 = pl.pallas_call(
        _tc_add_block,
        grid=(nsb, 3),
        in_specs=[
            pl.BlockSpec((_BS_A, D), lambda i, j: (j * nsb + i, 0)),
            pl.BlockSpec((_BS_A, D), lambda i, j: (i, 0)),
        ],
        out_specs=pl.BlockSpec((_BS_A, D), lambda i, j: (j * nsb + i, 0)),
        out_shape=jax.ShapeDtypeStruct((B * S, D), jnp.float32),
    )(flat, pe_s)

    nb = _TC2_ROWS // _BS_B
    base = _TC1_ROWS // _BS_B
    buf2 = pl.pallas_call(
        _tc_add_block_alias,
        grid=(nb,),
        in_specs=[
            pl.BlockSpec((_BS_B, D), lambda i: (base + i, 0)),
            pl.BlockSpec((_BS_B, D), lambda i: (i, 0)),
            pl.BlockSpec(memory_space=pl.ANY),
        ],
        out_specs=pl.BlockSpec((_BS_B, D), lambda i: (base + i, 0)),
        out_shape=jax.ShapeDtypeStruct((B * S, D), jnp.float32),
        input_output_aliases={2: 0},
    )(flat, pe_s, buf1)

    out = lax.dynamic_update_slice(buf2, sc_out, (_SC_ROW0, 0))
    return out.reshape(B, S, D)


# speedup vs baseline: 4.6469x; 1.4499x over previous
import jax
import jax.numpy as jnp
from jax.experimental import pallas as pl


def _add_pe_kernel(seq_ref, pe_ref, out_ref):
    out_ref[...] = seq_ref[...] + pe_ref[...]


def kernel(sequence, pe):
    B, S, D = sequence.shape
    BS = 2048
    while S % BS:
        BS //= 2
    nsb = S // BS
    flat = sequence.reshape(B * S, D)
    out = pl.pallas_call(
        _add_pe_kernel,
        grid=(nsb, B),
        in_specs=[
            pl.BlockSpec((BS, D), lambda i, j: (j * nsb + i, 0)),
            pl.BlockSpec((BS, D), lambda i, j: (i, 0)),
        ],
        out_specs=pl.BlockSpec((BS, D), lambda i, j: (j * nsb + i, 0)),
        out_shape=jax.ShapeDtypeStruct((B * S, D), sequence.dtype),
    )(flat, pe[:S])
    return out.reshape(B, S, D)
